# baseline (speedup 1.0000x reference)
import functools

import jax
import jax.numpy as jnp
from jax import lax
from jax.experimental import pallas as pl
from jax.experimental.pallas import tpu as pltpu
from jax.experimental.pallas import tpu_sc as plsc

_LANES = 16


def _embed_sum_sc(idx_flat, emb_table, batch, window, dim):
    nc, ns = 2, 16
    nw = nc * ns
    bpw = batch // nw
    slab = window * bpw

    mesh = plsc.VectorSubcoreMesh(core_axis_name="c", subcore_axis_name="s")

    @functools.partial(
        pl.kernel,
        out_type=jax.ShapeDtypeStruct((batch, dim), jnp.float32),
        mesh=mesh,
        compiler_params=pltpu.CompilerParams(use_tc_tiling_on_sc=False),
        scratch_types=[
            pltpu.VMEM((slab,), jnp.int32),
            pltpu.VMEM((window, bpw, dim), jnp.float32),
            pltpu.VMEM((bpw, dim), jnp.float32),
            pltpu.SemaphoreType.DMA,
        ],
    )
    def k(idx_hbm, table_hbm, out_hbm, idx_v, rows_v, acc_v, sem):
        wid = lax.axis_index("s") * nc + lax.axis_index("c")
        base = wid * bpw
        pltpu.sync_copy(idx_hbm.at[pl.ds(wid * slab, slab)], idx_v)
        copies = []
        for w in range(window):
            copies.append(
                pltpu.async_copy(
                    table_hbm.at[idx_v.at[pl.ds(w * bpw, bpw)]],
                    rows_v.at[w],
                    sem,
                )
            )
        for c in copies:
            c.wait()

        def body(b, carry):
            for c in range(dim // _LANES):
                sl = pl.ds(c * _LANES, _LANES)
                s = rows_v[0, b, sl]
                for w in range(1, window):
                    s = s + rows_v[w, b, sl]
                acc_v[b, sl] = s
            return carry

        lax.fori_loop(0, bpw, body, 0)
        pltpu.sync_copy(acc_v, out_hbm.at[pl.ds(base, bpw)])

    return k(idx_flat, emb_table)


def _mlp_tc(embeds, w1, b1, w2, b2, window):
    b, d = embeds.shape
    h = w1.shape[0]
    v = w2.shape[0]
    bv = 2048
    grid = pl.cdiv(v, bv)
    inv_win = 1.0 / float(window)

    def body(emb_ref, w1_ref, b1_ref, w2_ref, b2_ref, out_ref, h_ref):
        j = pl.program_id(0)

        @pl.when(j == 0)
        def _():
            hh = lax.dot_general(
                emb_ref[...], w1_ref[...], (((1,), (1,)), ((), ())),
                preferred_element_type=jnp.float32,
            )
            hh = (hh + b1_ref[...]) * inv_win
            h_ref[...] = jnp.maximum(hh, 0.0)

        out = lax.dot_general(
            h_ref[...], w2_ref[...], (((1,), (1,)), ((), ())),
            preferred_element_type=jnp.float32,
        )
        out_ref[...] = jnp.maximum(out + b2_ref[...], 0.0)

    return pl.pallas_call(
        body,
        grid=(grid,),
        in_specs=[
            pl.BlockSpec((b, d), lambda j: (0, 0)),
            pl.BlockSpec((h, d), lambda j: (0, 0)),
            pl.BlockSpec((1, h), lambda j: (0, 0)),
            pl.BlockSpec((bv, h), lambda j: (j, 0)),
            pl.BlockSpec((1, bv), lambda j: (0, j)),
        ],
        out_specs=pl.BlockSpec((b, bv), lambda j: (0, j)),
        out_shape=jax.ShapeDtypeStruct((b, v), jnp.float32),
        scratch_shapes=[pltpu.VMEM((b, h), jnp.float32)],
    )(embeds, w1, b1.reshape(1, h), w2, b2.reshape(1, v))


def kernel(inputs, emb_table, W1, b1, W2, b2):
    batch, window = inputs.shape
    dim = emb_table.shape[1]
    nw = 32
    bpw = batch // nw
    idx_flat = (
        inputs.reshape(nw, bpw, window).transpose(0, 2, 1).reshape(-1)
    )
    embeds = _embed_sum_sc(idx_flat, emb_table, batch, window, dim)
    return _mlp_tc(embeds, W1, b1, W2, b2, window)


# speedup vs baseline: 2.1252x; 2.1252x over previous
import functools

import jax
import jax.numpy as jnp
from jax import lax
from jax.experimental import pallas as pl
from jax.experimental.pallas import tpu as pltpu
from jax.experimental.pallas import tpu_sc as plsc

_LANES = 16


def _emb_w1_tc(table_t, w1, block_v):
    d, v = table_t.shape
    h = w1.shape[0]
    grid = pl.cdiv(v, block_v)

    def body(tt_ref, w1_ref, g_ref):
        g_ref[...] = lax.dot_general(
            tt_ref[...], w1_ref[...], (((0,), (1,)), ((), ())),
            preferred_element_type=jnp.float32,
        )

    return pl.pallas_call(
        body,
        grid=(grid,),
        in_specs=[
            pl.BlockSpec((d, block_v), lambda j: (0, j)),
            pl.BlockSpec((h, d), lambda j: (0, 0)),
        ],
        out_specs=pl.BlockSpec((block_v, h), lambda j: (j, 0)),
        out_shape=jax.ShapeDtypeStruct((v, h), jnp.float32),
    )(table_t, w1)


def _gather_sum_act_sc(idx_flat, g, b1, batch, window, hidden, inv_win):
    nc, ns = 2, 16
    nw = nc * ns
    bpw = batch // nw
    slab = window * bpw

    mesh = plsc.VectorSubcoreMesh(core_axis_name="c", subcore_axis_name="s")

    @functools.partial(
        pl.kernel,
        out_type=jax.ShapeDtypeStruct((batch, hidden), jnp.float32),
        mesh=mesh,
        scratch_types=[
            pltpu.VMEM((slab,), jnp.int32),
            pltpu.VMEM((window, bpw, hidden), jnp.float32),
            pltpu.VMEM((bpw, hidden), jnp.float32),
            pltpu.VMEM((hidden,), jnp.float32),
            pltpu.SemaphoreType.DMA,
        ],
    )
    def k(idx_hbm, g_hbm, b1_hbm, out_hbm, idx_v, rows_v, acc_v, b1_v, sem):
        wid = lax.axis_index("s") * nc + lax.axis_index("c")
        base = wid * bpw
        pltpu.sync_copy(b1_hbm, b1_v)
        pltpu.sync_copy(idx_hbm.at[pl.ds(wid * slab, slab)], idx_v)
        copies = []
        for w in range(window):
            copies.append(
                pltpu.async_copy(
                    g_hbm.at[idx_v.at[pl.ds(w * bpw, bpw)]],
                    rows_v.at[w],
                    sem,
                )
            )
        for c in copies:
            c.wait()

        def body(b, carry):
            for c in range(hidden // _LANES):
                sl = pl.ds(c * _LANES, _LANES)
                s = rows_v[0, b, sl]
                for w in range(1, window):
                    s = s + rows_v[w, b, sl]
                s = (s + b1_v[sl]) * inv_win
                acc_v[b, sl] = jnp.maximum(s, 0.0)
            return carry

        lax.fori_loop(0, bpw, body, 0)
        pltpu.sync_copy(acc_v, out_hbm.at[pl.ds(base, bpw)])

    return k(idx_flat, g, b1)


def _out_proj_tc(h_act, w2, b2, block_v):
    b, hid = h_act.shape
    v = w2.shape[0]
    grid = pl.cdiv(v, block_v)

    def body(h_ref, w2_ref, b2_ref, out_ref):
        acc = lax.dot_general(
            w2_ref[...], h_ref[...], (((1,), (1,)), ((), ())),
            preferred_element_type=jnp.float32,
        )
        out_ref[...] = jnp.maximum(acc + b2_ref[...], 0.0)

    return pl.pallas_call(
        body,
        grid=(grid,),
        in_specs=[
            pl.BlockSpec((b, hid), lambda j: (0, 0)),
            pl.BlockSpec((block_v, hid), lambda j: (j, 0)),
            pl.BlockSpec((block_v, 1), lambda j: (j, 0)),
        ],
        out_specs=pl.BlockSpec((block_v, b), lambda j: (j, 0)),
        out_shape=jax.ShapeDtypeStruct((v, b), jnp.float32),
    )(h_act, w2, b2.reshape(v, 1))


def kernel(inputs, emb_table, W1, b1, W2, b2):
    batch, window = inputs.shape
    hidden = W1.shape[0]
    nw = 32
    bpw = batch // nw
    idx_flat = (
        inputs.reshape(nw, bpw, window).transpose(0, 2, 1).reshape(-1)
    )
    g = _emb_w1_tc(emb_table.T, W1, 2048)
    h_act = _gather_sum_act_sc(
        idx_flat, g, b1, batch, window, hidden, 1.0 / float(window)
    )
    out_t = _out_proj_tc(h_act, W2, b2, 2048)
    return out_t.T


# speedup vs baseline: 2.6609x; 1.2521x over previous
import functools

import jax
import jax.numpy as jnp
from jax import lax
from jax.experimental import pallas as pl
from jax.experimental.pallas import tpu as pltpu
from jax.experimental.pallas import tpu_sc as plsc

_LANES = 16


def _emb_w1_tc(table_t, w1, block_v):
    d, v = table_t.shape
    h = w1.shape[0]
    grid = pl.cdiv(v, block_v)

    def body(tt_ref, w1_ref, g_ref):
        g_ref[...] = lax.dot_general(
            tt_ref[...].astype(jnp.bfloat16),
            w1_ref[...].astype(jnp.bfloat16),
            (((0,), (1,)), ((), ())),
            preferred_element_type=jnp.float32,
        )

    return pl.pallas_call(
        body,
        grid=(grid,),
        in_specs=[
            pl.BlockSpec((d, block_v), lambda j: (0, j)),
            pl.BlockSpec((h, d), lambda j: (0, 0)),
        ],
        out_specs=pl.BlockSpec((block_v, h), lambda j: (j, 0)),
        out_shape=jax.ShapeDtypeStruct((v, h), jnp.float32),
    )(table_t, w1)


def _gather_sum_act_sc(idx_flat, g, b1, batch, window, hidden, inv_win):
    nc, ns = 2, 16
    nw = nc * ns
    bpw = batch // nw
    slab = window * bpw

    mesh = plsc.VectorSubcoreMesh(core_axis_name="c", subcore_axis_name="s")

    @functools.partial(
        pl.kernel,
        out_type=jax.ShapeDtypeStruct((batch, hidden), jnp.float32),
        mesh=mesh,
        scratch_types=[
            pltpu.VMEM((slab,), jnp.int32),
            pltpu.VMEM((window, bpw, hidden), jnp.float32),
            pltpu.VMEM((bpw, hidden), jnp.float32),
            pltpu.VMEM((hidden,), jnp.float32),
            pltpu.SemaphoreType.DMA,
        ],
    )
    def k(idx_hbm, g_hbm, b1_hbm, out_hbm, idx_v, rows_v, acc_v, b1_v, sem):
        wid = lax.axis_index("s") * nc + lax.axis_index("c")
        base = wid * bpw
        pltpu.sync_copy(b1_hbm, b1_v)
        pltpu.sync_copy(idx_hbm.at[pl.ds(wid * slab, slab)], idx_v)
        copies = []
        for w in range(window):
            copies.append(
                pltpu.async_copy(
                    g_hbm.at[idx_v.at[pl.ds(w * bpw, bpw)]],
                    rows_v.at[w],
                    sem,
                )
            )
        for c in copies:
            c.wait()

        def body(b, carry):
            for c in range(hidden // _LANES):
                sl = pl.ds(c * _LANES, _LANES)
                s = rows_v[0, b, sl]
                for w in range(1, window):
                    s = s + rows_v[w, b, sl]
                s = (s + b1_v[sl]) * inv_win
                acc_v[b, sl] = jnp.maximum(s, 0.0)
            return carry

        lax.fori_loop(0, bpw, body, 0)
        pltpu.sync_copy(acc_v, out_hbm.at[pl.ds(base, bpw)])

    return k(idx_flat, g, b1)


def _out_proj_tc(h_act, w2, b2, block_v):
    b, hid = h_act.shape
    v = w2.shape[0]
    grid = pl.cdiv(v, block_v)

    def body(h_ref, w2_ref, b2_ref, out_ref):
        acc = lax.dot_general(
            w2_ref[...], h_ref[...], (((1,), (1,)), ((), ())),
            preferred_element_type=jnp.float32,
        )
        out_ref[...] = jnp.maximum(acc + b2_ref[...][:, None], 0.0)

    return pl.pallas_call(
        body,
        grid=(grid,),
        in_specs=[
            pl.BlockSpec((b, hid), lambda j: (0, 0)),
            pl.BlockSpec((block_v, hid), lambda j: (j, 0)),
            pl.BlockSpec((block_v,), lambda j: (j,)),
        ],
        out_specs=pl.BlockSpec((block_v, b), lambda j: (j, 0)),
        out_shape=jax.ShapeDtypeStruct((v, b), jnp.float32),
    )(h_act, w2, b2)


def kernel(inputs, emb_table, W1, b1, W2, b2):
    batch, window = inputs.shape
    hidden = W1.shape[0]
    nw = 32
    bpw = batch // nw
    idx_flat = (
        inputs.reshape(nw, bpw, window).transpose(0, 2, 1).reshape(-1)
    )
    g = _emb_w1_tc(emb_table.T, W1, 2048)
    h_act = _gather_sum_act_sc(
        idx_flat, g, b1, batch, window, hidden, 1.0 / float(window)
    )
    out_t = _out_proj_tc(h_act, W2, b2, 2048)
    return out_t.T


# speedup vs baseline: 2.9852x; 1.1219x over previous
import functools

import jax
import jax.numpy as jnp
from jax import lax
from jax.experimental import pallas as pl
from jax.experimental.pallas import tpu as pltpu
from jax.experimental.pallas import tpu_sc as plsc

_LANES = 16


def _emb_w1_tc(table_t, w1, block_v):
    d, v = table_t.shape
    h = w1.shape[0]
    grid = pl.cdiv(v, block_v)

    def body(tt_ref, w1_ref, g_ref):
        g_ref[...] = lax.dot_general(
            tt_ref[...].astype(jnp.bfloat16),
            w1_ref[...].astype(jnp.bfloat16),
            (((0,), (1,)), ((), ())),
            preferred_element_type=jnp.float32,
        )

    return pl.pallas_call(
        body,
        grid=(grid,),
        in_specs=[
            pl.BlockSpec((d, block_v), lambda j: (0, j)),
            pl.BlockSpec((h, d), lambda j: (0, 0)),
        ],
        out_specs=pl.BlockSpec((block_v, h), lambda j: (j, 0)),
        out_shape=jax.ShapeDtypeStruct((v, h), jnp.float32),
    )(table_t, w1)


def _gather_sum_act_sc(idx_flat, g, b1, batch, window, hidden, inv_win):
    nc, ns = 2, 16
    nw = nc * ns
    bpw = batch // nw
    slab = window * bpw

    mesh = plsc.VectorSubcoreMesh(core_axis_name="c", subcore_axis_name="s")

    @functools.partial(
        pl.kernel,
        out_type=jax.ShapeDtypeStruct((batch, hidden), jnp.float32),
        mesh=mesh,
        scratch_types=[
            pltpu.VMEM((slab,), jnp.int32),
            pltpu.VMEM((window, bpw, hidden), jnp.float32),
            pltpu.VMEM((bpw, hidden), jnp.float32),
            pltpu.VMEM((hidden,), jnp.float32),
            pltpu.SemaphoreType.DMA,
        ],
    )
    def k(idx_hbm, g_hbm, b1_hbm, out_hbm, idx_v, rows_v, acc_v, b1_v, sem):
        wid = lax.axis_index("s") * nc + lax.axis_index("c")
        base = wid * bpw
        pltpu.sync_copy(b1_hbm, b1_v)
        pltpu.sync_copy(idx_hbm.at[pl.ds(wid * slab, slab)], idx_v)
        copies = []
        for w in range(window):
            copies.append(
                pltpu.async_copy(
                    g_hbm.at[idx_v.at[pl.ds(w * bpw, bpw)]],
                    rows_v.at[w],
                    sem,
                )
            )
        for c in copies:
            c.wait()

        def body(b, carry):
            for c in range(hidden // _LANES):
                sl = pl.ds(c * _LANES, _LANES)
                s = rows_v[0, b, sl]
                for w in range(1, window):
                    s = s + rows_v[w, b, sl]
                s = (s + b1_v[sl]) * inv_win
                acc_v[b, sl] = jnp.maximum(s, 0.0)
            return carry

        lax.fori_loop(0, bpw, body, 0)
        pltpu.sync_copy(acc_v, out_hbm.at[pl.ds(base, bpw)])

    return k(idx_flat, g, b1)


def _out_proj_tc(h_act, w2, b2, block_v):
    b, hid = h_act.shape
    v = w2.shape[0]
    grid = pl.cdiv(v, block_v)

    def body(h_ref, w2_ref, b2_ref, out_ref):
        acc = lax.dot_general(
            w2_ref[...], h_ref[...], (((1,), (1,)), ((), ())),
            preferred_element_type=jnp.float32,
        )
        out_ref[...] = jnp.maximum(acc + b2_ref[...][:, None], 0.0)

    return pl.pallas_call(
        body,
        grid=(grid,),
        in_specs=[
            pl.BlockSpec((b, hid), lambda j: (0, 0)),
            pl.BlockSpec((block_v, hid), lambda j: (j, 0)),
            pl.BlockSpec((block_v,), lambda j: (j,)),
        ],
        out_specs=pl.BlockSpec((block_v, b), lambda j: (j, 0)),
        out_shape=jax.ShapeDtypeStruct((v, b), jnp.float32),
    )(h_act, w2, b2)


def kernel(inputs, emb_table, W1, b1, W2, b2):
    batch, window = inputs.shape
    hidden = W1.shape[0]
    nw = 32
    bpw = batch // nw
    idx_flat = (
        inputs.reshape(nw, bpw, window).transpose(0, 2, 1).reshape(-1)
    )
    g = _emb_w1_tc(emb_table.T, W1, 8192)
    h_act = _gather_sum_act_sc(
        idx_flat, g, b1, batch, window, hidden, 1.0 / float(window)
    )
    out_t = _out_proj_tc(h_act, W2, b2, 4096)
    return out_t.T


# speedup vs baseline: 2.9960x; 1.0036x over previous
import functools

import jax
import jax.numpy as jnp
from jax import lax
from jax.experimental import pallas as pl
from jax.experimental.pallas import tpu as pltpu
from jax.experimental.pallas import tpu_sc as plsc

_LANES = 16


def _emb_w1_tc(table_t, w1, block_v):
    d, v = table_t.shape
    h = w1.shape[0]
    grid = pl.cdiv(v, block_v)

    def body(tt_ref, w1_ref, g_ref):
        g_ref[...] = lax.dot_general(
            tt_ref[...].astype(jnp.bfloat16),
            w1_ref[...].astype(jnp.bfloat16),
            (((0,), (1,)), ((), ())),
            preferred_element_type=jnp.float32,
        )

    return pl.pallas_call(
        body,
        grid=(grid,),
        in_specs=[
            pl.BlockSpec((d, block_v), lambda j: (0, j)),
            pl.BlockSpec((h, d), lambda j: (0, 0)),
        ],
        out_specs=pl.BlockSpec((block_v, h), lambda j: (j, 0)),
        out_shape=jax.ShapeDtypeStruct((v, h), jnp.float32),
    )(table_t, w1)


def _gather_sum_act_sc(idx_flat, g, b1, batch, window, hidden, inv_win):
    nc, ns = 2, 16
    nw = nc * ns
    bpw = batch // nw
    slab = window * bpw

    mesh = plsc.VectorSubcoreMesh(core_axis_name="c", subcore_axis_name="s")

    @functools.partial(
        pl.kernel,
        out_type=jax.ShapeDtypeStruct((batch, hidden), jnp.float32),
        mesh=mesh,
        scratch_types=[
            pltpu.VMEM((slab,), jnp.int32),
            pltpu.VMEM((window, bpw, hidden), jnp.float32),
            pltpu.VMEM((bpw, hidden), jnp.float32),
            pltpu.VMEM((hidden,), jnp.float32),
            pltpu.SemaphoreType.DMA,
        ],
    )
    def k(idx_hbm, g_hbm, b1_hbm, out_hbm, idx_v, rows_v, acc_v, b1_v, sem):
        wid = lax.axis_index("s") * nc + lax.axis_index("c")
        base = wid * bpw
        pltpu.sync_copy(b1_hbm, b1_v)
        pltpu.sync_copy(idx_hbm.at[pl.ds(wid * slab, slab)], idx_v)
        copies = []
        for w in range(window):
            copies.append(
                pltpu.async_copy(
                    g_hbm.at[idx_v.at[pl.ds(w * bpw, bpw)]],
                    rows_v.at[w],
                    sem,
                )
            )
        for c in copies:
            c.wait()

        def body(b, carry):
            for c in range(hidden // _LANES):
                sl = pl.ds(c * _LANES, _LANES)
                s = rows_v[0, b, sl]
                for w in range(1, window):
                    s = s + rows_v[w, b, sl]
                s = (s + b1_v[sl]) * inv_win
                acc_v[b, sl] = jnp.maximum(s, 0.0)
            return carry

        lax.fori_loop(0, bpw, body, 0)
        pltpu.sync_copy(acc_v, out_hbm.at[pl.ds(base, bpw)])

    return k(idx_flat, g, b1)


def _out_proj_tc(h_act, w2, b2, block_v):
    b, hid = h_act.shape
    v = w2.shape[0]
    grid = pl.cdiv(v, block_v)

    def body(h_ref, w2_ref, b2_ref, out_ref):
        acc = lax.dot_general(
            w2_ref[...], h_ref[...], (((1,), (1,)), ((), ())),
            preferred_element_type=jnp.float32,
        )
        out_ref[...] = jnp.maximum(acc + b2_ref[...][:, None], 0.0)

    return pl.pallas_call(
        body,
        grid=(grid,),
        in_specs=[
            pl.BlockSpec((b, hid), lambda j: (0, 0)),
            pl.BlockSpec((block_v, hid), lambda j: (j, 0)),
            pl.BlockSpec((block_v,), lambda j: (j,)),
        ],
        out_specs=pl.BlockSpec((block_v, b), lambda j: (j, 0)),
        out_shape=jax.ShapeDtypeStruct((v, b), jnp.float32),
        compiler_params=pltpu.CompilerParams(
            vmem_limit_bytes=110 * 1024 * 1024,
        ),
    )(h_act, w2, b2)


def kernel(inputs, emb_table, W1, b1, W2, b2):
    batch, window = inputs.shape
    hidden = W1.shape[0]
    nw = 32
    bpw = batch // nw
    idx_flat = (
        inputs.reshape(nw, bpw, window).transpose(0, 2, 1).reshape(-1)
    )
    g = _emb_w1_tc(emb_table.T, W1, 8192)
    h_act = _gather_sum_act_sc(
        idx_flat, g, b1, batch, window, hidden, 1.0 / float(window)
    )
    out_t = _out_proj_tc(h_act, W2, b2, 6144)
    return out_t.T


# speedup vs baseline: 3.0268x; 1.0103x over previous
import functools

import jax
import jax.numpy as jnp
from jax import lax
from jax.experimental import pallas as pl
from jax.experimental.pallas import tpu as pltpu
from jax.experimental.pallas import tpu_sc as plsc

_LANES = 16


def _emb_w1_tc(table_t, w1, block_v):
    d, v = table_t.shape
    h = w1.shape[0]
    grid = pl.cdiv(v, block_v)

    def body(tt_ref, w1_ref, g_ref):
        g_ref[...] = lax.dot_general(
            tt_ref[...].astype(jnp.bfloat16),
            w1_ref[...].astype(jnp.bfloat16),
            (((0,), (1,)), ((), ())),
            preferred_element_type=jnp.float32,
        )

    return pl.pallas_call(
        body,
        grid=(grid,),
        in_specs=[
            pl.BlockSpec((d, block_v), lambda j: (0, j)),
            pl.BlockSpec((h, d), lambda j: (0, 0)),
        ],
        out_specs=pl.BlockSpec((block_v, h), lambda j: (j, 0)),
        out_shape=jax.ShapeDtypeStruct((v, h), jnp.float32),
    )(table_t, w1)


def _gather_sum_act_sc(idx_flat, g, b1, batch, window, hidden, inv_win):
    nc, ns = 2, 16
    nw = nc * ns
    bpw = batch // nw
    slab = window * bpw

    mesh = plsc.VectorSubcoreMesh(core_axis_name="c", subcore_axis_name="s")

    @functools.partial(
        pl.kernel,
        out_type=jax.ShapeDtypeStruct((batch, hidden), jnp.float32),
        mesh=mesh,
        scratch_types=[
            pltpu.VMEM((slab,), jnp.int32),
            pltpu.VMEM((window, bpw, hidden), jnp.float32),
            pltpu.VMEM((bpw, hidden), jnp.float32),
            pltpu.VMEM((hidden,), jnp.float32),
            pltpu.SemaphoreType.DMA,
        ],
    )
    def k(idx_hbm, g_hbm, b1_hbm, out_hbm, idx_v, rows_v, acc_v, b1_v, sem):
        wid = lax.axis_index("s") * nc + lax.axis_index("c")
        base = wid * bpw
        pltpu.sync_copy(b1_hbm, b1_v)
        pltpu.sync_copy(idx_hbm.at[pl.ds(wid * slab, slab)], idx_v)
        copies = []
        for w in range(window):
            copies.append(
                pltpu.async_copy(
                    g_hbm.at[idx_v.at[pl.ds(w * bpw, bpw)]],
                    rows_v.at[w],
                    sem,
                )
            )
        for c in copies:
            c.wait()

        def body(b, carry):
            for c in range(hidden // _LANES):
                sl = pl.ds(c * _LANES, _LANES)
                s = rows_v[0, b, sl]
                for w in range(1, window):
                    s = s + rows_v[w, b, sl]
                s = (s + b1_v[sl]) * inv_win
                acc_v[b, sl] = jnp.maximum(s, 0.0)
            return carry

        lax.fori_loop(0, bpw, body, 0)
        pltpu.sync_copy(acc_v, out_hbm.at[pl.ds(base, bpw)])

    return k(idx_flat, g, b1)


def _out_proj_tc(h_act, w2, b2, block_v):
    b, hid = h_act.shape
    v = w2.shape[0]
    grid = pl.cdiv(v, block_v)

    def body(h_ref, w2_ref, b2_ref, out_ref):
        acc = lax.dot_general(
            w2_ref[...], h_ref[...], (((1,), (1,)), ((), ())),
            preferred_element_type=jnp.float32,
        )
        out_ref[...] = jnp.maximum(acc + b2_ref[...][:, None], 0.0)

    return pl.pallas_call(
        body,
        grid=(grid,),
        in_specs=[
            pl.BlockSpec((b, hid), lambda j: (0, 0)),
            pl.BlockSpec((block_v, hid), lambda j: (j, 0)),
            pl.BlockSpec((block_v,), lambda j: (j,)),
        ],
        out_specs=pl.BlockSpec((block_v, b), lambda j: (j, 0)),
        out_shape=jax.ShapeDtypeStruct((v, b), jnp.float32),
        compiler_params=pltpu.CompilerParams(
            vmem_limit_bytes=110 * 1024 * 1024,
        ),
    )(h_act, w2, b2)


def kernel(inputs, emb_table, W1, b1, W2, b2):
    batch, window = inputs.shape
    hidden = W1.shape[0]
    nw = 32
    bpw = batch // nw
    idx_flat = (
        inputs.reshape(nw, bpw, window).transpose(0, 2, 1).reshape(-1)
    )
    g = _emb_w1_tc(emb_table.T, W1, 16384)
    h_act = _gather_sum_act_sc(
        idx_flat, g, b1, batch, window, hidden, 1.0 / float(window)
    )
    out_t = _out_proj_tc(h_act, W2, b2, 6144)
    return out_t.T


# speedup vs baseline: 3.0457x; 1.0062x over previous
import functools

import jax
import jax.numpy as jnp
from jax import lax
from jax.experimental import pallas as pl
from jax.experimental.pallas import tpu as pltpu
from jax.experimental.pallas import tpu_sc as plsc

_LANES = 16


def _emb_w1_tc(table_t, w1, block_v):
    d, v = table_t.shape
    h = w1.shape[0]
    grid = pl.cdiv(v, block_v)

    def body(tt_ref, w1_ref, g_ref):
        g_ref[...] = lax.dot_general(
            tt_ref[...].astype(jnp.bfloat16),
            w1_ref[...].astype(jnp.bfloat16),
            (((0,), (1,)), ((), ())),
            preferred_element_type=jnp.float32,
        )

    return pl.pallas_call(
        body,
        grid=(grid,),
        in_specs=[
            pl.BlockSpec((d, block_v), lambda j: (0, j)),
            pl.BlockSpec((h, d), lambda j: (0, 0)),
        ],
        out_specs=pl.BlockSpec((block_v, h), lambda j: (j, 0)),
        out_shape=jax.ShapeDtypeStruct((v, h), jnp.float32),
    )(table_t, w1)


def _gather_sum_act_sc(idx_flat, g, b1, batch, window, hidden, inv_win):
    nc, ns = 2, 16
    nw = nc * ns
    bpw = batch // nw
    slab = window * bpw

    mesh = plsc.VectorSubcoreMesh(core_axis_name="c", subcore_axis_name="s")

    @functools.partial(
        pl.kernel,
        out_type=jax.ShapeDtypeStruct((batch, hidden), jnp.float32),
        mesh=mesh,
        scratch_types=[
            pltpu.VMEM((slab,), jnp.int32),
            pltpu.VMEM((window, bpw, hidden), jnp.float32),
            pltpu.VMEM((bpw, hidden), jnp.float32),
            pltpu.VMEM((hidden,), jnp.float32),
            pltpu.SemaphoreType.DMA,
        ],
    )
    def k(idx_hbm, g_hbm, b1_hbm, out_hbm, idx_v, rows_v, acc_v, b1_v, sem):
        wid = lax.axis_index("s") * nc + lax.axis_index("c")
        base = wid * bpw
        pltpu.sync_copy(b1_hbm, b1_v)
        pltpu.sync_copy(idx_hbm.at[pl.ds(wid * slab, slab)], idx_v)
        copies = []
        for w in range(window):
            copies.append(
                pltpu.async_copy(
                    g_hbm.at[idx_v.at[pl.ds(w * bpw, bpw)]],
                    rows_v.at[w],
                    sem,
                )
            )
        for c in copies:
            c.wait()

        def body(b, carry):
            for c in range(hidden // _LANES):
                sl = pl.ds(c * _LANES, _LANES)
                s = rows_v[0, b, sl]
                for w in range(1, window):
                    s = s + rows_v[w, b, sl]
                s = (s + b1_v[sl]) * inv_win
                acc_v[b, sl] = jnp.maximum(s, 0.0)
            return carry

        lax.fori_loop(0, bpw, body, 0)
        pltpu.sync_copy(acc_v, out_hbm.at[pl.ds(base, bpw)])

    return k(idx_flat, g, b1)


def _out_proj_tc(h_act, w2, b2, block_v):
    b, hid = h_act.shape
    v = w2.shape[0]
    grid = pl.cdiv(v, block_v)

    def body(h_ref, w2_ref, b2_ref, out_ref):
        acc = lax.dot_general(
            w2_ref[...], h_ref[...], (((1,), (1,)), ((), ())),
            preferred_element_type=jnp.float32,
        )
        out_ref[...] = jnp.maximum(acc + b2_ref[...][:, None], 0.0)

    return pl.pallas_call(
        body,
        grid=(grid,),
        in_specs=[
            pl.BlockSpec((b, hid), lambda j: (0, 0)),
            pl.BlockSpec((block_v, hid), lambda j: (j, 0)),
            pl.BlockSpec((block_v,), lambda j: (j,)),
        ],
        out_specs=pl.BlockSpec((block_v, b), lambda j: (j, 0)),
        out_shape=jax.ShapeDtypeStruct((v, b), jnp.float32),
        compiler_params=pltpu.CompilerParams(
            vmem_limit_bytes=110 * 1024 * 1024,
        ),
    )(h_act, w2, b2)


def kernel(inputs, emb_table, W1, b1, W2, b2):
    batch, window = inputs.shape
    hidden = W1.shape[0]
    nw = 32
    bpw = batch // nw
    idx_flat = (
        inputs.reshape(nw, bpw, window).transpose(0, 2, 1).reshape(-1)
    )
    g = _emb_w1_tc(emb_table.T, W1, 32768)
    h_act = _gather_sum_act_sc(
        idx_flat, g, b1, batch, window, hidden, 1.0 / float(window)
    )
    out_t = _out_proj_tc(h_act, W2, b2, 6144)
    return out_t.T
